# Initial kernel scaffold; baseline (speedup 1.0000x reference)
import jax
import jax.numpy as jnp
from jax.experimental import pallas as pl


def kernel(token_ids, embedding_matrix):
    raise NotImplementedError("write your pallas kernel here")


# speedup vs baseline: 1.6824x; 1.6824x over previous
import jax
import jax.numpy as jnp
from jax import lax
from jax.experimental import pallas as pl
from jax.experimental.pallas import tpu as pltpu
from jax.experimental.pallas import tpu_sc as plsc

_NC = 2
_NS = 16
_NW = _NC * _NS

_B, _S = 16384, 50
_TOTAL = _B * _S
_BLK = 128
_NBLK = _TOTAL // _BLK
_PER_W = _NBLK // _NW
_D = 64


def _body(idx_hbm, tbl_hbm, out_hbm, idx_v, rows_v, gsem):
    wid = lax.axis_index("s") * _NC + lax.axis_index("c")
    base = wid * _PER_W
    pltpu.sync_copy(idx_hbm.at[pl.ds(base, _PER_W)], idx_v)

    def step(i, carry):
        pltpu.async_copy(tbl_hbm.at[idx_v.at[i]], rows_v, gsem).wait()
        pltpu.sync_copy(rows_v, out_hbm.at[pl.ds((base + i) * _BLK, _BLK)])
        return carry

    lax.fori_loop(0, _PER_W, step, 0)


_sc_gather = pl.kernel(
    _body,
    out_type=jax.ShapeDtypeStruct((_TOTAL, _D), jnp.float32),
    mesh=plsc.VectorSubcoreMesh(core_axis_name="c", subcore_axis_name="s"),
    scratch_types=[
        pltpu.VMEM((_PER_W, _BLK), jnp.int32),
        pltpu.VMEM((_BLK, _D), jnp.float32),
        pltpu.SemaphoreType.DMA,
    ],
    compiler_params=pltpu.CompilerParams(use_tc_tiling_on_sc=False),
)


def kernel(token_ids, embedding_matrix):
    idx = token_ids.reshape(_NBLK, _BLK).astype(jnp.int32)
    out = _sc_gather(idx, embedding_matrix)
    return out.reshape(_B, _S, _D)


# speedup vs baseline: 1.8762x; 1.1152x over previous
import jax
import jax.numpy as jnp
from jax import lax
from jax.experimental import pallas as pl
from jax.experimental.pallas import tpu as pltpu
from jax.experimental.pallas import tpu_sc as plsc

_NC = 2
_NS = 16
_NW = _NC * _NS

_B, _S = 16384, 50
_TOTAL = _B * _S
_BLK = 128
_NBLK = _TOTAL // _BLK
_PER_W = _NBLK // _NW
_D = 64

_NBUF = 8
_LAG = 4


def _body(idx_hbm, tbl_hbm, out_hbm, idx_v, rows_v, gsem, osem):
    wid = lax.axis_index("s") * _NC + lax.axis_index("c")
    base = wid * _PER_W
    pltpu.sync_copy(idx_hbm.at[pl.ds(base, _PER_W)], idx_v)

    def start_gather(t, slot):
        pltpu.async_copy(tbl_hbm.at[idx_v.at[t]], rows_v.at[slot], gsem.at[slot])

    def wait_gather(slot):
        pltpu.make_async_copy(
            tbl_hbm.at[idx_v.at[0]], rows_v.at[slot], gsem.at[slot]
        ).wait()

    def start_out(t, slot):
        pltpu.async_copy(
            rows_v.at[slot], out_hbm.at[pl.ds((base + t) * _BLK, _BLK)],
            osem.at[slot],
        )

    def wait_out(slot):
        pltpu.make_async_copy(
            rows_v.at[slot], out_hbm.at[pl.ds(0, _BLK)], osem.at[slot]
        ).wait()

    for u in range(_LAG):
        start_gather(u, u)
    for t in range(_LAG):
        start_gather(t + _LAG, t + _LAG)
        wait_gather(t % _NBUF)
        start_out(t, t % _NBUF)

    n_main = (_PER_W - _LAG - _NBUF) // _NBUF

    def outer(o, carry):
        t0 = _LAG + o * _NBUF
        for j in range(_NBUF):
            t = t0 + j
            bg = j % _NBUF
            bo = (_LAG + j) % _NBUF
            wait_out(bg)
            start_gather(t + _LAG, bg)
            wait_gather(bo)
            start_out(t, bo)
        return carry

    lax.fori_loop(0, n_main, outer, 0)

    for t in range(_LAG + n_main * _NBUF, _PER_W):
        bg = (t + _LAG) % _NBUF
        bo = t % _NBUF
        if t + _LAG < _PER_W:
            wait_out(bg)
            start_gather(t + _LAG, bg)
        wait_gather(bo)
        start_out(t, bo)
    for slot in range(_NBUF):
        wait_out(slot)


_sc_gather = pl.kernel(
    _body,
    out_type=jax.ShapeDtypeStruct((_TOTAL, _D), jnp.float32),
    mesh=plsc.VectorSubcoreMesh(core_axis_name="c", subcore_axis_name="s"),
    scratch_types=[
        pltpu.VMEM((_PER_W, _BLK), jnp.int32),
        pltpu.VMEM((_NBUF, _BLK, _D), jnp.float32),
        pltpu.SemaphoreType.DMA((_NBUF,)),
        pltpu.SemaphoreType.DMA((_NBUF,)),
    ],
    compiler_params=pltpu.CompilerParams(use_tc_tiling_on_sc=False),
)


def kernel(token_ids, embedding_matrix):
    idx = token_ids.reshape(_NBLK, _BLK).astype(jnp.int32)
    out = _sc_gather(idx, embedding_matrix)
    return out.reshape(_B, _S, _D)
